# baseline (speedup 1.0000x reference)
import jax
import jax.numpy as jnp
from jax.experimental import pallas as pl
from jax.experimental.pallas import tpu as pltpu

_SEQ_LENS = (2048, 1792, 1536, 1280, 1024, 896, 768, 512)
_D = 1024
_MAX_L = 2048
_MAX_PAD = _MAX_L - min(_SEQ_LENS)


def _body(*refs):
    in_refs = refs[:8]
    out_ref = refs[8]
    bufs = refs[9:17]
    zero_ref = refs[17]
    in_sems = refs[18]
    out_sems = refs[19]
    pad_sems = refs[20]

    zero_ref[...] = jnp.zeros(zero_ref.shape, zero_ref.dtype)

    in_copies = [
        pltpu.make_async_copy(in_refs[i], bufs[i], in_sems.at[i])
        for i in range(8)
    ]
    for c in in_copies:
        c.start()

    pad_copies = []
    for i, L in enumerate(_SEQ_LENS):
        pad = _MAX_L - L
        if pad:
            c = pltpu.make_async_copy(
                zero_ref.at[:, pl.ds(0, pad), :],
                out_ref.at[pl.ds(i, 1), pl.ds(L, pad), :],
                pad_sems.at[i],
            )
            c.start()
            pad_copies.append(c)

    out_copies = []
    for i in sorted(range(8), key=lambda i: _SEQ_LENS[i]):
        L = _SEQ_LENS[i]
        in_copies[i].wait()
        c = pltpu.make_async_copy(
            bufs[i],
            out_ref.at[pl.ds(i, 1), pl.ds(0, L), :],
            out_sems.at[i],
        )
        c.start()
        out_copies.append(c)

    for c in out_copies:
        c.wait()
    for c in pad_copies:
        c.wait()


def kernel(seq0, seq1, seq2, seq3, seq4, seq5, seq6, seq7):
    seqs = (seq0, seq1, seq2, seq3, seq4, seq5, seq6, seq7)
    out_shape = jax.ShapeDtypeStruct((8, _MAX_L, _D), seq0.dtype)
    return pl.pallas_call(
        _body,
        in_specs=[pl.BlockSpec(memory_space=pl.ANY)] * 8,
        out_specs=pl.BlockSpec(memory_space=pl.ANY),
        out_shape=out_shape,
        scratch_shapes=(
            [pltpu.VMEM((1, L, _D), jnp.float32) for L in _SEQ_LENS]
            + [
                pltpu.VMEM((1, _MAX_PAD, _D), jnp.float32),
                pltpu.SemaphoreType.DMA((8,)),
                pltpu.SemaphoreType.DMA((8,)),
                pltpu.SemaphoreType.DMA((8,)),
            ]
        ),
    )(*seqs)


# speedup vs baseline: 1.0589x; 1.0589x over previous
import jax
import jax.numpy as jnp
from jax.experimental import pallas as pl
from jax.experimental.pallas import tpu as pltpu

_SEQ_LENS = (2048, 1792, 1536, 1280, 1024, 896, 768, 512)
_D = 1024
_MAX_L = 2048
_MAX_PAD = _MAX_L - min(_SEQ_LENS)
_CHUNK = 512
_N_CHUNKS = sum(-(-L // _CHUNK) for L in _SEQ_LENS)


def _body(*refs):
    in_refs = refs[:8]
    out_ref = refs[8]
    bufs = refs[9:17]
    zero_ref = refs[17]
    in_sems = refs[18]
    out_sems = refs[19]
    pad_sems = refs[20]

    zero_ref[...] = jnp.zeros(zero_ref.shape, zero_ref.dtype)

    chunks = []
    for t in range(_MAX_L // _CHUNK):
        for i, L in enumerate(_SEQ_LENS):
            r0 = t * _CHUNK
            rows = min(_CHUNK, L - r0)
            if rows > 0:
                chunks.append((i, r0, rows))

    in_copies = []
    for ci, (i, r0, rows) in enumerate(chunks):
        c = pltpu.make_async_copy(
            in_refs[i].at[:, pl.ds(r0, rows), :],
            bufs[i].at[:, pl.ds(r0, rows), :],
            in_sems.at[ci],
        )
        c.start()
        in_copies.append(c)

    pad_copies = []
    for i, L in enumerate(_SEQ_LENS):
        pad = _MAX_L - L
        if pad:
            c = pltpu.make_async_copy(
                zero_ref.at[:, pl.ds(0, pad), :],
                out_ref.at[pl.ds(i, 1), pl.ds(L, pad), :],
                pad_sems.at[i],
            )
            c.start()
            pad_copies.append(c)

    out_copies = []
    for ci, (i, r0, rows) in enumerate(chunks):
        in_copies[ci].wait()
        c = pltpu.make_async_copy(
            bufs[i].at[:, pl.ds(r0, rows), :],
            out_ref.at[pl.ds(i, 1), pl.ds(r0, rows), :],
            out_sems.at[ci],
        )
        c.start()
        out_copies.append(c)

    for c in out_copies:
        c.wait()
    for c in pad_copies:
        c.wait()


def kernel(seq0, seq1, seq2, seq3, seq4, seq5, seq6, seq7):
    seqs = (seq0, seq1, seq2, seq3, seq4, seq5, seq6, seq7)
    out_shape = jax.ShapeDtypeStruct((8, _MAX_L, _D), seq0.dtype)
    return pl.pallas_call(
        _body,
        in_specs=[pl.BlockSpec(memory_space=pl.ANY)] * 8,
        out_specs=pl.BlockSpec(memory_space=pl.ANY),
        out_shape=out_shape,
        scratch_shapes=(
            [pltpu.VMEM((1, L, _D), jnp.float32) for L in _SEQ_LENS]
            + [
                pltpu.VMEM((1, _MAX_PAD, _D), jnp.float32),
                pltpu.SemaphoreType.DMA((_N_CHUNKS,)),
                pltpu.SemaphoreType.DMA((_N_CHUNKS,)),
                pltpu.SemaphoreType.DMA((8,)),
            ]
        ),
    )(*seqs)
